# Initial kernel scaffold; baseline (speedup 1.0000x reference)
import jax
import jax.numpy as jnp
from jax.experimental import pallas as pl


def kernel(x, y, memory_x, memory_y, eye):
    raise NotImplementedError("write your pallas kernel here")


# speedup vs baseline: 2.6587x; 2.6587x over previous
import functools

import jax
import jax.numpy as jnp
from jax import lax
from jax.experimental import pallas as pl
from jax.experimental.pallas import tpu as pltpu

NUM_CLASSES = 10
K = 5
MEMORY_SIZE = 50000
N_SAMP = 10000
N_PAD = 10240
QB = 256

_SENTINEL = 3.4028235e38


def _knn_block_kernel(xf_ref, xn_ref, memT_ref, aux_ref, eye_ref, out_ref):
    xq = xf_ref[...]
    memT = memT_ref[...]
    aux = aux_ref[...]
    xn = xn_ref[...]
    yn = aux[0:1, :]
    lab = aux[1:2, :]

    mm = jnp.dot(xq, memT)
    v = (xn + yn) - 2.0 * mm

    col = lax.broadcasted_iota(jnp.int32, (QB, N_PAD), 1)
    for _ in range(K):
        m = jnp.min(v, axis=1, keepdims=True)
        masked_iota = jnp.where(v == m, col, jnp.int32(N_PAD))
        idx = jnp.min(masked_iota, axis=1, keepdims=True)
        v = jnp.where(col == idx, _SENTINEL, v)

    sel = (v == _SENTINEL).astype(jnp.float32)

    cls16 = lax.broadcasted_iota(jnp.int32, (16, N_PAD), 0)
    onehotT = (lab.astype(jnp.int32) == cls16).astype(jnp.float32)
    counts = lax.dot_general(sel, onehotT,
                             (((1,), (1,)), ((), ())))

    maxc = jnp.max(counts, axis=1, keepdims=True)
    cls_row = lax.broadcasted_iota(jnp.int32, (QB, 16), 1)
    pred = jnp.min(jnp.where(counts == maxc, cls_row, jnp.int32(16)),
                   axis=1, keepdims=True)
    oh = (cls_row == pred).astype(jnp.float32)
    out16 = jnp.dot(oh, eye_ref[...])
    out_ref[...] = out16[:, :NUM_CLASSES]


def _run_tc(xf, xn, memT, aux, eye16):
    n = xf.shape[0]
    grid = n // QB
    return pl.pallas_call(
        _knn_block_kernel,
        grid=(grid,),
        in_specs=[
            pl.BlockSpec((QB, 16), lambda i: (i, 0)),
            pl.BlockSpec((QB, 1), lambda i: (i, 0)),
            pl.BlockSpec((16, N_PAD), lambda i: (0, 0)),
            pl.BlockSpec((8, N_PAD), lambda i: (0, 0)),
            pl.BlockSpec((16, 16), lambda i: (0, 0)),
        ],
        out_specs=pl.BlockSpec((QB, NUM_CLASSES), lambda i: (i, 0)),
        out_shape=jax.ShapeDtypeStruct((n, NUM_CLASSES), jnp.float32),
        compiler_params=pltpu.CompilerParams(
            dimension_semantics=("parallel",)),
    )(xf, xn, memT, aux, eye16)


def kernel(x, y, memory_x, memory_y, eye):
    b, c, h, w = x.shape
    xf = jnp.transpose(x, (0, 2, 3, 1)).reshape(b * h * w, c)
    n = xf.shape[0]

    n_samp = min(MEMORY_SIZE, N_SAMP)
    mem_idx = jax.random.randint(jax.random.key(1234), (n_samp,), 0, n,
                                 dtype=jnp.int32)

    sample = jnp.take(memory_x, mem_idx, axis=0)
    labels = jnp.take(memory_y[:, 0], mem_idx, axis=0)

    xn = jnp.sum(xf ** 2, axis=1).reshape(-1, 1)
    yn = jnp.sum(sample ** 2, axis=1)

    pad = N_PAD - n_samp
    memT = jnp.concatenate(
        [sample, jnp.zeros((pad, c), jnp.float32)], axis=0).T
    yn_p = jnp.concatenate([yn, jnp.full((pad,), jnp.inf, jnp.float32)])
    lab_p = jnp.concatenate([labels, jnp.zeros((pad,), jnp.int32)])
    aux = jnp.zeros((8, N_PAD), jnp.float32)
    aux = aux.at[0, :].set(yn_p)
    aux = aux.at[1, :].set(lab_p.astype(jnp.float32))

    eye16 = jnp.zeros((16, 16), jnp.float32).at[:NUM_CLASSES, :NUM_CLASSES].set(eye)

    out2d = _run_tc(xf, xn, memT, aux, eye16)
    return jnp.transpose(out2d.reshape(b, h, w, NUM_CLASSES), (0, 3, 1, 2))
